# baseline (speedup 1.0000x reference)
import functools

import jax
import jax.numpy as jnp
import numpy as np
from jax.experimental import pallas as pl


_TWO_PI = 2.0 * np.pi
_DIR_OFFSET = 0.78539


def _body(bp_ref, rt_ref, an_ref, out_ref):
    bp = bp_ref[...]
    rt = rt_ref[...]
    bp6 = bp[:, 6:7]
    rt6 = rt[:, 6:7]
    an6 = an_ref[:, 6:7]

    s1 = jnp.sin(bp6) * jnp.cos(rt6)
    s2 = jnp.cos(bp6) * jnp.sin(rt6)

    x = rt6 + an6 - _DIR_OFFSET
    offset_rot = x - jnp.floor(x / _TWO_PI) * _TWO_PI
    d = jnp.clip(jnp.floor(offset_rot / np.pi), 0.0, 1.0)

    out_ref[...] = jnp.concatenate(
        [bp[:, :6], s1, rt[:, :6], s2, 1.0 - d, d], axis=1)


@functools.partial(jax.jit, static_argnames=("block_rows",))
def _run(bp, rt, an, block_rows):
    rows = bp.shape[0]
    grid = (rows // block_rows,)
    in_spec = pl.BlockSpec((block_rows, 7), lambda i: (i, 0))
    out_spec = pl.BlockSpec((block_rows, 16), lambda i: (i, 0))
    return pl.pallas_call(
        _body,
        grid=grid,
        in_specs=[in_spec, in_spec, in_spec],
        out_specs=out_spec,
        out_shape=jax.ShapeDtypeStruct((rows, 16), bp.dtype),
    )(bp, rt, an)


def kernel(box_preds, reg_targets, anchors):
    B, N, C = box_preds.shape
    rows = B * N
    bp = box_preds.reshape(rows, C)
    rt = reg_targets.reshape(rows, C)
    an = anchors.reshape(rows, C)
    block_rows = 4224
    out = _run(bp, rt, an, block_rows)
    return out.reshape(B, N, 16)


# speedup vs baseline: 3.6199x; 3.6199x over previous
import functools

import jax
import jax.numpy as jnp
import numpy as np
from jax.experimental import pallas as pl


_TWO_PI = 2.0 * np.pi
_DIR_OFFSET = 0.78539


def _body(bp_ref, rt_ref, an_ref, out_ref):
    bpT = jnp.transpose(bp_ref[...])
    rtT = jnp.transpose(rt_ref[...])
    anT = jnp.transpose(an_ref[...])

    bp6 = bpT[6:7, :]
    rt6 = rtT[6:7, :]
    an6 = anT[6:7, :]

    u = jnp.sin(bp6 + rt6)
    v = jnp.sin(bp6 - rt6)
    s1 = (u + v) * 0.5
    s2 = (u - v) * 0.5

    x = rt6 + an6 - _DIR_OFFSET
    m = x - jnp.floor(x / _TWO_PI) * _TWO_PI
    d = jnp.clip(jnp.floor(m / np.pi), 0.0, 1.0)

    outT = jnp.concatenate(
        [bpT[:6], s1, rtT[:6], s2, 1.0 - d, d], axis=0)
    out_ref[...] = jnp.transpose(outT)


@functools.partial(jax.jit, static_argnames=("block_rows",))
def _run(bp, rt, an, block_rows):
    rows = bp.shape[0]
    grid = (rows // block_rows,)
    in_spec = pl.BlockSpec((block_rows, 7), lambda i: (i, 0))
    out_spec = pl.BlockSpec((block_rows, 16), lambda i: (i, 0))
    return pl.pallas_call(
        _body,
        grid=grid,
        in_specs=[in_spec, in_spec, in_spec],
        out_specs=out_spec,
        out_shape=jax.ShapeDtypeStruct((rows, 16), bp.dtype),
    )(bp, rt, an)


def kernel(box_preds, reg_targets, anchors):
    B, N, C = box_preds.shape
    rows = B * N
    bp = box_preds.reshape(rows, C)
    rt = reg_targets.reshape(rows, C)
    an = anchors.reshape(rows, C)
    block_rows = 2112
    out = _run(bp, rt, an, block_rows)
    return out.reshape(B, N, 16)


# speedup vs baseline: 4.2817x; 1.1828x over previous
import functools

import jax
import jax.numpy as jnp
import numpy as np
from jax.experimental import pallas as pl


_TWO_PI = 2.0 * np.pi
_DIR_OFFSET = 0.78539


def _body(bp_ref, rt_ref, an_ref, out_ref):
    bpT = jnp.transpose(bp_ref[...])
    rtT = jnp.transpose(rt_ref[...])
    anT = jnp.transpose(an_ref[...])

    bp6 = bpT[6:7, :]
    rt6 = rtT[6:7, :]
    an6 = anT[6:7, :]

    suv = jnp.sin(jnp.concatenate([bp6 + rt6, bp6 - rt6], axis=0))
    u = suv[0:1, :]
    v = suv[1:2, :]
    s1 = (u + v) * 0.5
    s2 = (u - v) * 0.5

    x = rt6 + an6 - _DIR_OFFSET
    m = x - jnp.floor(x / _TWO_PI) * _TWO_PI
    d = jnp.clip(jnp.floor(m / np.pi), 0.0, 1.0)

    outT = jnp.concatenate(
        [bpT[:6], s1, rtT[:6], s2, 1.0 - d, d], axis=0)
    out_ref[...] = jnp.transpose(outT)


@functools.partial(jax.jit, static_argnames=("block_rows",))
def _run(bp, rt, an, block_rows):
    rows = bp.shape[0]
    grid = (rows // block_rows,)
    in_spec = pl.BlockSpec((block_rows, 7), lambda i: (i, 0))
    out_spec = pl.BlockSpec((block_rows, 16), lambda i: (i, 0))
    return pl.pallas_call(
        _body,
        grid=grid,
        in_specs=[in_spec, in_spec, in_spec],
        out_specs=out_spec,
        out_shape=jax.ShapeDtypeStruct((rows, 16), bp.dtype),
    )(bp, rt, an)


def kernel(box_preds, reg_targets, anchors):
    B, N, C = box_preds.shape
    rows = B * N
    bp = box_preds.reshape(rows, C)
    rt = reg_targets.reshape(rows, C)
    an = anchors.reshape(rows, C)
    block_rows = 4224
    out = _run(bp, rt, an, block_rows)
    return out.reshape(B, N, 16)
